# Initial kernel scaffold; baseline (speedup 1.0000x reference)
import jax
import jax.numpy as jnp
from jax.experimental import pallas as pl


def kernel(data, segment_ids, num_segments):
    raise NotImplementedError("write your pallas kernel here")


# speedup vs baseline: 3.5794x; 3.5794x over previous
import functools

import jax
import jax.numpy as jnp
from jax import lax
from jax.experimental import pallas as pl
from jax.experimental.pallas import tpu as pltpu
from jax.experimental.pallas import tpu_sc as plsc

N_ROWS = 320000
D = 128
S = 10000
NC = 2
NS = 16
NW = NC * NS
R = N_ROWS // NW
C = 80
NCHUNK = R // C
ZR = 200
NZ = S // ZR
NZ_ITER = (NZ + NS - 1) // NS


def _sc_body(data_hbm, ids_hbm, zeros_hbm, part_hbm, dbuf, ibuf, acc):
    c = lax.axis_index("c")
    s = lax.axis_index("s")
    wid = c * NS + s

    def zero_chunk(k, carry):
        j = s + k * NS

        @pl.when(j < NZ)
        def _():
            pltpu.sync_copy(zeros_hbm, acc.at[pl.ds(j * ZR, ZR)])

        return carry

    lax.fori_loop(0, NZ_ITER, zero_chunk, 0)
    plsc.subcore_barrier()

    base0 = wid * R

    def chunk(k, carry):
        b = base0 + k * C
        pltpu.sync_copy(ids_hbm.at[pl.ds(b, C)], ibuf)
        pltpu.sync_copy(data_hbm.at[pl.ds(b, C)], dbuf)
        pltpu.sync_copy(dbuf, acc.at[ibuf], add=True)
        return carry

    lax.fori_loop(0, NCHUNK, chunk, 0)
    plsc.subcore_barrier()

    def wout_chunk(k, carry):
        j = s + k * NS

        @pl.when(j < NZ)
        def _():
            pltpu.sync_copy(acc.at[pl.ds(j * ZR, ZR)],
                            part_hbm.at[pl.ds(c * S + j * ZR, ZR)])

        return carry

    lax.fori_loop(0, NZ_ITER, wout_chunk, 0)


@jax.jit
def _sc_segment_sum(data, ids, zeros):
    mesh = plsc.VectorSubcoreMesh(
        core_axis_name="c", subcore_axis_name="s",
        num_cores=NC, num_subcores=NS)
    f = pl.kernel(
        _sc_body,
        out_type=jax.ShapeDtypeStruct((NC * S, D), jnp.float32),
        mesh=mesh,
        scratch_types=[
            pltpu.VMEM((C, D), jnp.float32),
            pltpu.VMEM((C,), jnp.int32),
            pltpu.VMEM_SHARED((S, D), jnp.float32),
        ],
    )
    return f(data, ids, zeros)


def _combine_body(a_ref, b_ref, o_ref):
    o_ref[...] = a_ref[...] + b_ref[...]


@jax.jit
def _combine(part):
    a = part[:S]
    b = part[S:]
    blk = 1000
    return pl.pallas_call(
        _combine_body,
        grid=(S // blk,),
        in_specs=[pl.BlockSpec((blk, D), lambda i: (i, 0))] * 2,
        out_specs=pl.BlockSpec((blk, D), lambda i: (i, 0)),
        out_shape=jax.ShapeDtypeStruct((S, D), jnp.float32),
    )(a, b)


def kernel(data, segment_ids, num_segments):
    ids = segment_ids.astype(jnp.int32)
    zeros = jnp.zeros((ZR, D), jnp.float32)
    part = _sc_segment_sum(data, ids, zeros)
    return _combine(part)


# speedup vs baseline: 7.6033x; 2.1242x over previous
import jax
import jax.numpy as jnp
from jax import lax
from jax.experimental import pallas as pl
from jax.experimental.pallas import tpu as pltpu
from jax.experimental.pallas import tpu_sc as plsc

N_ROWS = 320000
D = 128
S = 10000
NC = 2
NS = 16
NW = NC * NS
R = N_ROWS // NW
C = 40
NCHUNK = R // C
NBUF = 5
NZCH = S // C
NZ_ITER = (NZCH + NS - 1) // NS


def _sc_body(data_hbm, ids_hbm, zeros_hbm, part_hbm, dbuf, ibuf, acc, *sems):
    semd = sems[:NBUF]
    semi = sems[NBUF:]
    c = lax.axis_index("c")
    s = lax.axis_index("s")
    wid = c * NS + s
    base0 = wid * R

    pltpu.sync_copy(zeros_hbm, dbuf.at[0])

    def zero_chunk(k, carry):
        j = s + k * NS

        @pl.when(j < NZCH)
        def _():
            pltpu.sync_copy(dbuf.at[0], acc.at[pl.ds(j * C, C)])

        return carry

    lax.fori_loop(0, NZ_ITER, zero_chunk, 0)
    plsc.subcore_barrier()

    for b in range(NBUF):
        off = base0 + b * C
        pltpu.async_copy(data_hbm.at[pl.ds(off, C)], dbuf.at[b], semd[b])
        pltpu.async_copy(ids_hbm.at[pl.ds(off, C)], ibuf.at[b], semi[b])

    def outer(g, carry):
        for b in range(NBUF):
            k = g * NBUF + b
            pltpu.make_async_copy(
                data_hbm.at[pl.ds(0, C)], dbuf.at[b], semd[b]).wait()
            pltpu.make_async_copy(
                ids_hbm.at[pl.ds(0, C)], ibuf.at[b], semi[b]).wait()
            pltpu.sync_copy(dbuf.at[b], acc.at[ibuf.at[b]], add=True)
            nk = k + NBUF

            @pl.when(nk < NCHUNK)
            def _():
                off = base0 + nk * C
                pltpu.async_copy(data_hbm.at[pl.ds(off, C)], dbuf.at[b],
                                 semd[b])
                pltpu.async_copy(ids_hbm.at[pl.ds(off, C)], ibuf.at[b],
                                 semi[b])

        return carry

    lax.fori_loop(0, NCHUNK // NBUF, outer, 0)
    plsc.subcore_barrier()

    def wout_chunk(k, carry):
        j = s + k * NS

        @pl.when(j < NZCH)
        def _():
            pltpu.sync_copy(acc.at[pl.ds(j * C, C)],
                            part_hbm.at[pl.ds(c * S + j * C, C)])

        return carry

    lax.fori_loop(0, NZ_ITER, wout_chunk, 0)


@jax.jit
def _sc_segment_sum(data, ids, zeros):
    mesh = plsc.VectorSubcoreMesh(
        core_axis_name="c", subcore_axis_name="s",
        num_cores=NC, num_subcores=NS)
    f = pl.kernel(
        _sc_body,
        out_type=jax.ShapeDtypeStruct((NC * S, D), jnp.float32),
        mesh=mesh,
        scratch_types=(
            [pltpu.VMEM((NBUF, C, D), jnp.float32),
             pltpu.VMEM((NBUF, C), jnp.int32),
             pltpu.VMEM_SHARED((S, D), jnp.float32)]
            + [pltpu.SemaphoreType.DMA] * (2 * NBUF)
        ),
    )
    return f(data, ids, zeros)


def _combine_body(a_ref, b_ref, o_ref):
    o_ref[...] = a_ref[...] + b_ref[...]


@jax.jit
def _combine(part):
    a = part[:S]
    b = part[S:]
    blk = 1000
    return pl.pallas_call(
        _combine_body,
        grid=(S // blk,),
        in_specs=[pl.BlockSpec((blk, D), lambda i: (i, 0))] * 2,
        out_specs=pl.BlockSpec((blk, D), lambda i: (i, 0)),
        out_shape=jax.ShapeDtypeStruct((S, D), jnp.float32),
    )(a, b)


def kernel(data, segment_ids, num_segments):
    ids = segment_ids.astype(jnp.int32)
    zeros = jnp.zeros((C, D), jnp.float32)
    part = _sc_segment_sum(data, ids, zeros)
    return _combine(part)


# speedup vs baseline: 8.1793x; 1.0758x over previous
import jax
import jax.numpy as jnp
from jax import lax
from jax.experimental import pallas as pl
from jax.experimental.pallas import tpu as pltpu
from jax.experimental.pallas import tpu_sc as plsc

N_ROWS = 320000
D = 128
S = 10000
S_HALF = S // 2
NC = 2
NS = 16
C = 128
CT = 16
NBUF = 4
MAXG = 40
ACC_R = S_HALF + 8
ZC = 40
NZCH = S_HALF // ZC
NZ_ITER = (NZCH + NS - 1) // NS
NBLK = N_ROWS // 16


def _sc_body(data_hbm, ids_hbm, zeros_hbm, out_hbm,
             dbuf, ibuf, ibuf2, dbt, ibt, ibt2, sbuf, zbuf, acc, *sems):
    semd = sems[:NBUF]
    semi = sems[NBUF:]
    c = lax.axis_index("c")
    s = lax.axis_index("s")
    base_seg = c * S_HALF

    pltpu.sync_copy(zeros_hbm, zbuf)

    def zero_chunk(k, carry):
        j = s + k * NS

        @pl.when(j < NZCH)
        def _():
            pltpu.sync_copy(zbuf, acc.at[pl.ds(j * ZC, ZC)])

        return carry

    lax.fori_loop(0, NZ_ITER, zero_chunk, 0)

    def probe(blk):
        pltpu.sync_copy(ids_hbm.at[pl.ds(blk * 16, 16)], sbuf)

    lo = jnp.int32(0)
    for step in [2 ** p for p in range(14, -1, -1)]:
        cand = lo + step
        candc = jnp.minimum(cand, NBLK - 1)
        probe(candc)
        take = (cand < NBLK) & (sbuf[...][0] < S_HALF)
        lo = jnp.where(take, cand, lo)
    probe(lo)
    below = jnp.where(sbuf[...] < S_HALF, 1, 0)
    cnt = below[0]
    for i in range(1, 16):
        cnt = cnt + below[i]
    split = lo * 16 + cnt

    up16 = jnp.bitwise_and(split + 15, jnp.int32(~15))
    dn16 = jnp.bitwise_and(split, jnp.int32(~15))
    start = jnp.where(c == 0, 0, dn16)
    end = jnp.where(c == 0, up16, N_ROWS)
    count = end - start
    per = (count >> 8) << 4
    mystart = pl.multiple_of(start + s * per, 16)
    myend = jnp.where(s == NS - 1, end, mystart + per)
    mylen = myend - mystart
    nfull = mylen >> 7
    ntail = jnp.bitwise_and(mylen, jnp.int32(127)) >> 4

    def fix_ids(v):
        rel = v - base_seg
        ok = (rel >= 0) & (rel < S_HALF)
        return jnp.where(ok, rel, S_HALF)

    plsc.subcore_barrier()

    def issue(slot, k):
        off = pl.multiple_of(mystart + k * C, 16)
        pltpu.async_copy(data_hbm.at[pl.ds(off, C)], dbuf.at[slot],
                         semd[slot])
        pltpu.async_copy(ids_hbm.at[pl.ds(off, C)], ibuf.at[slot],
                         semi[slot])

    for b in range(NBUF):
        pl.when(b < nfull)(lambda b=b: issue(b, jnp.int32(b)))

    def outer(g, carry):
        for b in range(NBUF):
            k = g * NBUF + b

            def do(b=b, k=k):
                pltpu.make_async_copy(
                    data_hbm.at[pl.ds(0, C)], dbuf.at[b], semd[b]).wait()
                pltpu.make_async_copy(
                    ids_hbm.at[pl.ds(0, C)], ibuf.at[b], semi[b]).wait()
                for v in range(C // 16):
                    ibuf2[b, pl.ds(v * 16, 16)] = fix_ids(
                        ibuf[b, pl.ds(v * 16, 16)])
                pltpu.sync_copy(dbuf.at[b], acc.at[ibuf2.at[b]], add=True)
                pl.when(k + NBUF < nfull)(lambda: issue(b, k + NBUF))

            pl.when(k < nfull)(do)
        return carry

    lax.fori_loop(0, MAXG, outer, 0)

    def tail_chunk(t, carry):
        off = pl.multiple_of(mystart + nfull * C + t * CT, 16)
        pltpu.sync_copy(data_hbm.at[pl.ds(off, CT)], dbt)
        pltpu.sync_copy(ids_hbm.at[pl.ds(off, CT)], ibt)
        ibt2[...] = fix_ids(ibt[...])
        pltpu.sync_copy(dbt, acc.at[ibt2], add=True)
        return carry

    lax.fori_loop(0, ntail, tail_chunk, 0)
    plsc.subcore_barrier()

    def wout_chunk(k, carry):
        j = s + k * NS

        @pl.when(j < NZCH)
        def _():
            pltpu.sync_copy(acc.at[pl.ds(j * ZC, ZC)],
                            out_hbm.at[pl.ds(base_seg + j * ZC, ZC)])

        return carry

    lax.fori_loop(0, NZ_ITER, wout_chunk, 0)


@jax.jit
def _sc_segment_sum(data, ids, zeros):
    mesh = plsc.VectorSubcoreMesh(
        core_axis_name="c", subcore_axis_name="s",
        num_cores=NC, num_subcores=NS)
    f = pl.kernel(
        _sc_body,
        out_type=jax.ShapeDtypeStruct((S, D), jnp.float32),
        mesh=mesh,
        scratch_types=(
            [pltpu.VMEM((NBUF, C, D), jnp.float32),
             pltpu.VMEM((NBUF, C), jnp.int32),
             pltpu.VMEM((NBUF, C), jnp.int32),
             pltpu.VMEM((CT, D), jnp.float32),
             pltpu.VMEM((CT,), jnp.int32),
             pltpu.VMEM((CT,), jnp.int32),
             pltpu.VMEM((16,), jnp.int32),
             pltpu.VMEM((ZC, D), jnp.float32),
             pltpu.VMEM_SHARED((ACC_R, D), jnp.float32)]
            + [pltpu.SemaphoreType.DMA] * (2 * NBUF)
        ),
    )
    return f(data, ids, zeros)


def kernel(data, segment_ids, num_segments):
    ids = segment_ids.astype(jnp.int32)
    zeros = jnp.zeros((ZC, D), jnp.float32)
    return _sc_segment_sum(data, ids, zeros)


# speedup vs baseline: 8.2123x; 1.0040x over previous
import jax
import jax.numpy as jnp
from jax import lax
from jax.experimental import pallas as pl
from jax.experimental.pallas import tpu as pltpu
from jax.experimental.pallas import tpu_sc as plsc

N_ROWS = 320000
D = 128
S = 10000
S_HALF = S // 2
NC = 2
NS = 16
C = 128
CT = 16
NBUF = 4
MAXG = 40
ACC_R = S_HALF + 8
ZC = 40
NZCH = S_HALF // ZC
NZ_ITER = (NZCH + NS - 1) // NS
NBLK = N_ROWS // 16


def _sc_body(data_hbm, ids_hbm, zeros_hbm, out_hbm,
             dbuf, ibuf, ibuf2, dbt, ibt, ibt2, sbuf, zbuf, acc, *sems):
    semd = sems[:NBUF]
    semi = sems[NBUF:]
    c = lax.axis_index("c")
    s = lax.axis_index("s")
    base_seg = c * S_HALF

    pltpu.sync_copy(zeros_hbm, zbuf)

    def zero_chunk(k, carry):
        j = s + k * NS

        @pl.when(j < NZCH)
        def _():
            pltpu.async_copy(zbuf, acc.at[pl.ds(j * ZC, ZC)], semi[0])

        return carry

    lax.fori_loop(0, NZ_ITER, zero_chunk, 0)

    def probe(blk):
        pltpu.sync_copy(ids_hbm.at[pl.ds(blk * 16, 16)], sbuf)

    lo = jnp.int32(0)
    for step in [2 ** p for p in range(14, -1, -1)]:
        cand = lo + step
        candc = jnp.minimum(cand, NBLK - 1)
        probe(candc)
        take = (cand < NBLK) & (sbuf[...][0] < S_HALF)
        lo = jnp.where(take, cand, lo)
    probe(lo)
    below = jnp.where(sbuf[...] < S_HALF, 1, 0)
    cnt = below[0]
    for i in range(1, 16):
        cnt = cnt + below[i]
    split = lo * 16 + cnt

    def zero_drain(k, carry):
        j = s + k * NS

        @pl.when(j < NZCH)
        def _():
            pltpu.make_async_copy(
                zbuf, acc.at[pl.ds(j * ZC, ZC)], semi[0]).wait()

        return carry

    lax.fori_loop(0, NZ_ITER, zero_drain, 0)

    up16 = jnp.bitwise_and(split + 15, jnp.int32(~15))
    dn16 = jnp.bitwise_and(split, jnp.int32(~15))
    start = jnp.where(c == 0, 0, dn16)
    end = jnp.where(c == 0, up16, N_ROWS)
    count = end - start
    per = (count >> 8) << 4
    rem = (count - (per << 4)) >> 4
    mystart = pl.multiple_of(
        start + s * per + jnp.minimum(s, rem) * 16, 16)
    mylen = per + jnp.where(s < rem, 16, 0)
    nfull = mylen >> 7
    ntail = jnp.bitwise_and(mylen, jnp.int32(127)) >> 4

    def fix_ids(v):
        rel = v - base_seg
        ok = (rel >= 0) & (rel < S_HALF)
        return jnp.where(ok, rel, S_HALF)

    plsc.subcore_barrier()

    def issue(slot, k):
        off = pl.multiple_of(mystart + k * C, 16)
        pltpu.async_copy(data_hbm.at[pl.ds(off, C)], dbuf.at[slot],
                         semd[slot])
        pltpu.async_copy(ids_hbm.at[pl.ds(off, C)], ibuf.at[slot],
                         semi[slot])

    for b in range(NBUF):
        pl.when(b < nfull)(lambda b=b: issue(b, jnp.int32(b)))

    def outer(g, carry):
        for b in range(NBUF):
            k = g * NBUF + b

            def do(b=b, k=k):
                pltpu.make_async_copy(
                    data_hbm.at[pl.ds(0, C)], dbuf.at[b], semd[b]).wait()
                pltpu.make_async_copy(
                    ids_hbm.at[pl.ds(0, C)], ibuf.at[b], semi[b]).wait()
                for v in range(C // 16):
                    ibuf2[b, pl.ds(v * 16, 16)] = fix_ids(
                        ibuf[b, pl.ds(v * 16, 16)])
                pltpu.sync_copy(dbuf.at[b], acc.at[ibuf2.at[b]], add=True)
                pl.when(k + NBUF < nfull)(lambda: issue(b, k + NBUF))

            pl.when(k < nfull)(do)
        return carry

    lax.fori_loop(0, MAXG, outer, 0)

    def tail_chunk(t, carry):
        off = pl.multiple_of(mystart + nfull * C + t * CT, 16)
        pltpu.sync_copy(data_hbm.at[pl.ds(off, CT)], dbt)
        pltpu.sync_copy(ids_hbm.at[pl.ds(off, CT)], ibt)
        ibt2[...] = fix_ids(ibt[...])
        pltpu.sync_copy(dbt, acc.at[ibt2], add=True)
        return carry

    lax.fori_loop(0, ntail, tail_chunk, 0)
    plsc.subcore_barrier()

    def wout_chunk(k, carry):
        j = s + k * NS

        @pl.when(j < NZCH)
        def _():
            pltpu.sync_copy(acc.at[pl.ds(j * ZC, ZC)],
                            out_hbm.at[pl.ds(base_seg + j * ZC, ZC)])

        return carry

    lax.fori_loop(0, NZ_ITER, wout_chunk, 0)


@jax.jit
def _sc_segment_sum(data, ids, zeros):
    mesh = plsc.VectorSubcoreMesh(
        core_axis_name="c", subcore_axis_name="s",
        num_cores=NC, num_subcores=NS)
    f = pl.kernel(
        _sc_body,
        out_type=jax.ShapeDtypeStruct((S, D), jnp.float32),
        mesh=mesh,
        scratch_types=(
            [pltpu.VMEM((NBUF, C, D), jnp.float32),
             pltpu.VMEM((NBUF, C), jnp.int32),
             pltpu.VMEM((NBUF, C), jnp.int32),
             pltpu.VMEM((CT, D), jnp.float32),
             pltpu.VMEM((CT,), jnp.int32),
             pltpu.VMEM((CT,), jnp.int32),
             pltpu.VMEM((16,), jnp.int32),
             pltpu.VMEM((ZC, D), jnp.float32),
             pltpu.VMEM_SHARED((ACC_R, D), jnp.float32)]
            + [pltpu.SemaphoreType.DMA] * (2 * NBUF)
        ),
    )
    return f(data, ids, zeros)


def kernel(data, segment_ids, num_segments):
    ids = segment_ids.astype(jnp.int32)
    zeros = jnp.zeros((ZC, D), jnp.float32)
    return _sc_segment_sum(data, ids, zeros)
